# baseline (speedup 1.0000x reference)
import functools

import jax
import jax.numpy as jnp
from jax import lax
from jax.experimental import pallas as pl
from jax.experimental.pallas import tpu as pltpu
from jax.experimental.pallas import tpu_sc as plsc


def _argmin_body(q_ref, k_ref, qsq_ref, ksq_ref, idx_ref, dist_ref, *, k_total, k_blk):
    j = pl.program_id(0)
    q = q_ref[...]
    k = k_ref[...]

    def _zero_oob():
        rows = lax.broadcasted_iota(jnp.int32, k.shape, 0) + j * k_blk
        return jnp.where(rows < k_total, k, 0.0)

    k = lax.cond(j == pl.num_programs(0) - 1, _zero_oob, lambda: k)
    s2 = lax.dot_general(
        q + q, k, (((1,), (1,)), ((), ())),
        preferred_element_type=jnp.float32,
        precision=lax.Precision.DEFAULT,
    )
    col_row = lax.broadcasted_iota(jnp.int32, (1, k_blk), 1) + j * k_blk
    ksq = jnp.where(col_row < k_total, ksq_ref[...], jnp.inf)
    qsq_all = qsq_ref[...]
    groups = k_blk // 128
    q_n = s2.shape[0]
    r_tile = 128
    lane = lax.broadcasted_iota(jnp.int32, (1, 128), 1).astype(jnp.float32)
    m_parts, bidx_parts = [], []
    for rt in range(0, q_n, r_tile):
        qsq = qsq_all[rt:rt + r_tile, :]
        cur_min = (qsq - s2[rt:rt + r_tile, 0:128]) + ksq[:, 0:128]
        cur_g = jnp.zeros_like(cur_min)
        for g in range(1, groups):
            x = (qsq - s2[rt:rt + r_tile, g * 128:(g + 1) * 128]) + ksq[
                :, g * 128:(g + 1) * 128
            ]
            upd = x < cur_min
            cur_g = jnp.where(upd, jnp.float32(g), cur_g)
            cur_min = jnp.where(upd, x, cur_min)
        m_t = jnp.min(cur_min, axis=1, keepdims=True)
        col_l = cur_g * 128.0 + lane
        bidx_parts.append(
            jnp.min(
                jnp.where(cur_min == m_t, col_l, jnp.float32(2**30)),
                axis=1,
                keepdims=True,
            )
        )
        m_parts.append(m_t)
    m = jnp.concatenate(m_parts, axis=0)
    bidx = jnp.concatenate(bidx_parts, axis=0).astype(jnp.int32) + j * k_blk

    @pl.when(j == 0)
    def _():
        idx_ref[...] = bidx
        dist_ref[...] = m

    @pl.when(j > 0)
    def _():
        prev = dist_ref[...]
        better = m < prev
        idx_ref[...] = jnp.where(better, bidx, idx_ref[...])
        dist_ref[...] = jnp.where(better, m, prev)


def _distance_argmin(queries, keys, q_sq, k_sq, k_blk=11264):
    q_n, d = queries.shape
    k_total = keys.shape[0]
    nb = pl.cdiv(k_total, k_blk)
    idx2, dist2 = pl.pallas_call(
        functools.partial(_argmin_body, k_total=k_total, k_blk=k_blk),
        grid=(nb,),
        in_specs=[
            pl.BlockSpec((q_n, d), lambda j: (0, 0)),
            pl.BlockSpec((k_blk, d), lambda j: (j, 0)),
            pl.BlockSpec((q_n, 1), lambda j: (0, 0)),
            pl.BlockSpec((1, k_blk), lambda j: (0, j)),
        ],
        out_specs=[
            pl.BlockSpec((q_n, 1), lambda j: (0, 0)),
            pl.BlockSpec((q_n, 1), lambda j: (0, 0)),
        ],
        out_shape=[
            jax.ShapeDtypeStruct((q_n, 1), jnp.int32),
            jax.ShapeDtypeStruct((q_n, 1), jnp.float32),
        ],
    )(queries, keys, q_sq, k_sq)
    return idx2.reshape(q_n), dist2.reshape(q_n)


def _sc_gather(keys, idx):
    b, d = idx.shape[0], keys.shape[1]
    n_workers = 32
    b_per_w = b // n_workers
    mesh = plsc.VectorSubcoreMesh(core_axis_name="c", subcore_axis_name="s")

    @functools.partial(
        pl.kernel,
        mesh=mesh,
        out_type=jax.ShapeDtypeStruct((b, d), jnp.float32),
        scratch_types=[
            pltpu.VMEM((b_per_w,), jnp.int32),
            pltpu.VMEM((b_per_w, d), jnp.float32),
            pltpu.SemaphoreType.DMA,
        ],
    )
    def k(keys_hbm, idx_hbm, out_hbm, idx_v, rows_v, sem):
        wid = lax.axis_index("s") * 2 + lax.axis_index("c")
        base = wid * b_per_w
        pltpu.sync_copy(idx_hbm.at[pl.ds(base, b_per_w)], idx_v)
        pltpu.async_copy(keys_hbm.at[idx_v], rows_v, sem).wait()
        pltpu.sync_copy(rows_v, out_hbm.at[pl.ds(base, b_per_w)])

    return k(keys, idx)


def kernel(queries, keys):
    q_sq = jnp.sum(queries * queries, axis=-1, keepdims=True)
    k_sq = jnp.sum(keys * keys, axis=-1)[None, :]
    best_idx, min_dists = _distance_argmin(queries, keys, q_sq, k_sq)
    best_vecs = _sc_gather(keys, best_idx)
    return best_idx, min_dists, best_vecs


# speedup vs baseline: 1.0003x; 1.0003x over previous
import functools

import jax
import jax.numpy as jnp
from jax import lax
from jax.experimental import pallas as pl
from jax.experimental.pallas import tpu as pltpu
from jax.experimental.pallas import tpu_sc as plsc


def _argmin_body(q_ref, k_ref, qsq_ref, ksq_ref, idx_ref, dist_ref, *, k_total, k_blk):
    j = pl.program_id(0)
    q = q_ref[...]
    k = k_ref[...]

    def _zero_oob():
        rows = lax.broadcasted_iota(jnp.int32, k.shape, 0) + j * k_blk
        return jnp.where(rows < k_total, k, 0.0)

    k = lax.cond(j == pl.num_programs(0) - 1, _zero_oob, lambda: k)
    s2 = lax.dot_general(
        q + q, k, (((1,), (1,)), ((), ())),
        preferred_element_type=jnp.float32,
        precision=lax.Precision.DEFAULT,
    )
    col_row = lax.broadcasted_iota(jnp.int32, (1, k_blk), 1) + j * k_blk
    ksq = jnp.where(col_row < k_total, ksq_ref[...], jnp.inf)
    qsq_all = qsq_ref[...]
    groups = k_blk // 128
    q_n = s2.shape[0]
    r_tile = 128
    lane = lax.broadcasted_iota(jnp.int32, (1, 128), 1).astype(jnp.float32)
    m_parts, bidx_parts = [], []
    for rt in range(0, q_n, r_tile):
        qsq = qsq_all[rt:rt + r_tile, :]
        cur_min = (qsq - s2[rt:rt + r_tile, 0:128]) + ksq[:, 0:128]
        cur_g = jnp.zeros_like(cur_min)
        for g in range(1, groups):
            x = (qsq - s2[rt:rt + r_tile, g * 128:(g + 1) * 128]) + ksq[
                :, g * 128:(g + 1) * 128
            ]
            upd = x < cur_min
            cur_g = jnp.where(upd, jnp.float32(g), cur_g)
            cur_min = jnp.where(upd, x, cur_min)
        m_t = jnp.min(cur_min, axis=1, keepdims=True)
        col_l = cur_g * 128.0 + lane
        bidx_parts.append(
            jnp.min(
                jnp.where(cur_min == m_t, col_l, jnp.float32(2**30)),
                axis=1,
                keepdims=True,
            )
        )
        m_parts.append(m_t)
    m = jnp.concatenate(m_parts, axis=0)
    bidx = jnp.concatenate(bidx_parts, axis=0).astype(jnp.int32) + j * k_blk

    @pl.when(j == 0)
    def _():
        idx_ref[...] = bidx
        dist_ref[...] = m

    @pl.when(j > 0)
    def _():
        prev = dist_ref[...]
        better = m < prev
        idx_ref[...] = jnp.where(better, bidx, idx_ref[...])
        dist_ref[...] = jnp.where(better, m, prev)


def _distance_argmin(queries, keys, q_sq, k_sq, k_blk=10240):
    q_n, d = queries.shape
    k_total = keys.shape[0]
    nb = pl.cdiv(k_total, k_blk)
    idx2, dist2 = pl.pallas_call(
        functools.partial(_argmin_body, k_total=k_total, k_blk=k_blk),
        grid=(nb,),
        in_specs=[
            pl.BlockSpec((q_n, d), lambda j: (0, 0)),
            pl.BlockSpec((k_blk, d), lambda j: (j, 0)),
            pl.BlockSpec((q_n, 1), lambda j: (0, 0)),
            pl.BlockSpec((1, k_blk), lambda j: (0, j)),
        ],
        out_specs=[
            pl.BlockSpec((q_n, 1), lambda j: (0, 0)),
            pl.BlockSpec((q_n, 1), lambda j: (0, 0)),
        ],
        out_shape=[
            jax.ShapeDtypeStruct((q_n, 1), jnp.int32),
            jax.ShapeDtypeStruct((q_n, 1), jnp.float32),
        ],
    )(queries, keys, q_sq, k_sq)
    return idx2.reshape(q_n), dist2.reshape(q_n)


def _sc_gather(keys, idx):
    b, d = idx.shape[0], keys.shape[1]
    n_workers = 32
    b_per_w = b // n_workers
    mesh = plsc.VectorSubcoreMesh(core_axis_name="c", subcore_axis_name="s")

    @functools.partial(
        pl.kernel,
        mesh=mesh,
        out_type=jax.ShapeDtypeStruct((b, d), jnp.float32),
        scratch_types=[
            pltpu.VMEM((b_per_w,), jnp.int32),
            pltpu.VMEM((b_per_w, d), jnp.float32),
            pltpu.SemaphoreType.DMA,
        ],
    )
    def k(keys_hbm, idx_hbm, out_hbm, idx_v, rows_v, sem):
        wid = lax.axis_index("s") * 2 + lax.axis_index("c")
        base = wid * b_per_w
        pltpu.sync_copy(idx_hbm.at[pl.ds(base, b_per_w)], idx_v)
        pltpu.async_copy(keys_hbm.at[idx_v], rows_v, sem).wait()
        pltpu.sync_copy(rows_v, out_hbm.at[pl.ds(base, b_per_w)])

    return k(keys, idx)


def kernel(queries, keys):
    q_sq = jnp.sum(queries * queries, axis=-1, keepdims=True)
    k_sq = jnp.sum(keys * keys, axis=-1)[None, :]
    best_idx, min_dists = _distance_argmin(queries, keys, q_sq, k_sq)
    best_vecs = _sc_gather(keys, best_idx)
    return best_idx, min_dists, best_vecs


# speedup vs baseline: 1.1145x; 1.1141x over previous
import functools

import jax
import jax.numpy as jnp
from jax import lax
from jax.experimental import pallas as pl
from jax.experimental.pallas import tpu as pltpu
from jax.experimental.pallas import tpu_sc as plsc


def _argmin_body(q_ref, k_ref, qsq_ref, ksq_ref, idx_ref, dist_ref, *, k_total, k_blk):
    j = pl.program_id(0)
    q = q_ref[...]
    k = k_ref[...]
    rows = lax.broadcasted_iota(jnp.int32, k.shape, 0) + j * k_blk
    k = jnp.where(rows < k_total, k, 0.0)
    s2 = lax.dot_general(
        q + q, k, (((1,), (1,)), ((), ())),
        preferred_element_type=jnp.float32,
        precision=lax.Precision.DEFAULT,
    )
    col_row = lax.broadcasted_iota(jnp.int32, (1, k_blk), 1) + j * k_blk
    ksq = jnp.where(col_row < k_total, ksq_ref[...], jnp.inf)
    qsq_all = qsq_ref[...]
    groups = k_blk // 128
    q_n = s2.shape[0]
    r_tile = 128
    lane = lax.broadcasted_iota(jnp.int32, (1, 128), 1).astype(jnp.float32)
    m_parts, bidx_parts = [], []
    for rt in range(0, q_n, r_tile):
        qsq = qsq_all[rt:rt + r_tile, :]
        cur_min = (qsq - s2[rt:rt + r_tile, 0:128]) + ksq[:, 0:128]
        cur_g = jnp.zeros_like(cur_min)
        for g in range(1, groups):
            x = (qsq - s2[rt:rt + r_tile, g * 128:(g + 1) * 128]) + ksq[
                :, g * 128:(g + 1) * 128
            ]
            upd = x < cur_min
            cur_g = jnp.where(upd, jnp.float32(g), cur_g)
            cur_min = jnp.where(upd, x, cur_min)
        m_t = jnp.min(cur_min, axis=1, keepdims=True)
        col_l = cur_g * 128.0 + lane
        bidx_parts.append(
            jnp.min(
                jnp.where(cur_min == m_t, col_l, jnp.float32(2**30)),
                axis=1,
                keepdims=True,
            )
        )
        m_parts.append(m_t)
    m = jnp.concatenate(m_parts, axis=0)
    bidx = jnp.concatenate(bidx_parts, axis=0).astype(jnp.int32) + j * k_blk

    @pl.when(j == 0)
    def _():
        idx_ref[...] = bidx
        dist_ref[...] = m

    @pl.when(j > 0)
    def _():
        prev = dist_ref[...]
        better = m < prev
        idx_ref[...] = jnp.where(better, bidx, idx_ref[...])
        dist_ref[...] = jnp.where(better, m, prev)


def _distance_argmin(queries, keys, q_sq, k_sq, k_blk=10240):
    q_n, d = queries.shape
    k_total = keys.shape[0]
    nb = pl.cdiv(k_total, k_blk)
    idx2, dist2 = pl.pallas_call(
        functools.partial(_argmin_body, k_total=k_total, k_blk=k_blk),
        grid=(nb,),
        in_specs=[
            pl.BlockSpec((q_n, d), lambda j: (0, 0)),
            pl.BlockSpec((k_blk, d), lambda j: (j, 0)),
            pl.BlockSpec((q_n, 1), lambda j: (0, 0)),
            pl.BlockSpec((1, k_blk), lambda j: (0, j)),
        ],
        out_specs=[
            pl.BlockSpec((q_n, 1), lambda j: (0, 0)),
            pl.BlockSpec((q_n, 1), lambda j: (0, 0)),
        ],
        out_shape=[
            jax.ShapeDtypeStruct((q_n, 1), jnp.int32),
            jax.ShapeDtypeStruct((q_n, 1), jnp.float32),
        ],
    )(queries, keys, q_sq, k_sq)
    return idx2.reshape(q_n), dist2.reshape(q_n)


def _sc_gather(keys, idx):
    b, d = idx.shape[0], keys.shape[1]
    n_workers = 32
    b_per_w = b // n_workers
    mesh = plsc.VectorSubcoreMesh(core_axis_name="c", subcore_axis_name="s")

    @functools.partial(
        pl.kernel,
        mesh=mesh,
        out_type=jax.ShapeDtypeStruct((b, d), jnp.float32),
        scratch_types=[
            pltpu.VMEM((b_per_w,), jnp.int32),
            pltpu.VMEM((b_per_w, d), jnp.float32),
            pltpu.SemaphoreType.DMA,
        ],
    )
    def k(keys_hbm, idx_hbm, out_hbm, idx_v, rows_v, sem):
        wid = lax.axis_index("s") * 2 + lax.axis_index("c")
        base = wid * b_per_w
        pltpu.sync_copy(idx_hbm.at[pl.ds(base, b_per_w)], idx_v)
        pltpu.async_copy(keys_hbm.at[idx_v], rows_v, sem).wait()
        pltpu.sync_copy(rows_v, out_hbm.at[pl.ds(base, b_per_w)])

    return k(keys, idx)


def kernel(queries, keys):
    q_sq = jnp.sum(queries * queries, axis=-1, keepdims=True)
    k_sq = jnp.sum(keys * keys, axis=-1)[None, :]
    best_idx, min_dists = _distance_argmin(queries, keys, q_sq, k_sq)
    best_vecs = _sc_gather(keys, best_idx)
    return best_idx, min_dists, best_vecs
